# baseline (speedup 1.0000x reference)
import jax
import jax.numpy as jnp
import numpy as np
from jax.experimental import pallas as pl
from jax.experimental.pallas import tpu as pltpu

_N, _L, _K = 128, 8192, 20
_RT = 8
_BL = 512
_SG = _BL // 128
_TINY = np.float32(np.finfo(np.float32).tiny)


def _threefry_bits(cnt):
    ks = (np.uint32(0), np.uint32(1), np.uint32(0x1BD11BDB))
    rot = (13, 15, 26, 6, 17, 29, 16, 24)

    x0 = jnp.zeros_like(cnt)
    x1 = cnt + ks[1]

    def rotl(v, d):
        return jax.lax.shift_left(v, np.uint32(d)) | jax.lax.shift_right_logical(
            v, np.uint32(32 - d))

    for i in range(5):
        rs = rot[:4] if i % 2 == 0 else rot[4:]
        for r in rs:
            x0 = x0 + x1
            x1 = rotl(x1, r)
            x1 = x0 ^ x1
        x0 = x0 + ks[(i + 1) % 3]
        x1 = x1 + ks[(i + 2) % 3] + np.uint32(i + 1)
    return x0 ^ x1


def _fused_kernel(t_ref, ab_ref, x0_ref, m_ref, c_ref, xt_ref):
    i0 = pl.program_id(0)
    i1 = pl.program_id(1)

    kio = jax.lax.broadcasted_iota(jnp.int32, (_K, 128), 0)
    lane20 = jax.lax.broadcasted_iota(jnp.uint32, (_K, 128), 1) * np.uint32(_K) \
        + kio.astype(jnp.uint32)

    for rr in range(_RT):
        n = i0 * _RT + rr
        ab = ab_ref[t_ref[n]]
        q = (1.0 - ab) / 20.0
        a = ab + q
        rowbase = n * (_L * _K) + i1 * (_BL * _K)

        for s in range(_SG):
            x0s = jnp.broadcast_to(x0_ref[rr, s * 128:(s + 1) * 128][None, :],
                                   (_K, 128))
            ms = jnp.broadcast_to(m_ref[rr, s * 128:(s + 1) * 128][None, :],
                                  (_K, 128)) != 0
            oh = x0s == kio
            c_like = jnp.where(
                ms, jnp.where(oh, a, q),
                jnp.where(oh, 1.0, 0.0)).astype(jnp.float32)

            c_ref[rr, s * 128:(s + 1) * 128, :] = c_like.T

            logits = jnp.log(c_like + 1e-8)

            cnt = jnp.uint32(rowbase + s * (128 * _K)) + lane20
            bits = _threefry_bits(cnt)
            fb = jax.lax.shift_right_logical(bits, np.uint32(9)) \
                | np.uint32(0x3F800000)
            f = jax.lax.bitcast_convert_type(fb, jnp.float32) - 1.0
            u = jnp.maximum(_TINY, f + _TINY)
            g = -jnp.log(-jnp.log(u))

            s_val = logits + g
            xt_ref[rr, s * 128:(s + 1) * 128] = \
                jnp.argmax(s_val, axis=0).astype(jnp.int32)


@jax.jit
def kernel(x_0, mask_generate, t, alpha_bars):
    m_i32 = mask_generate.astype(jnp.int32)

    c_t, x_t = pl.pallas_call(
        _fused_kernel,
        grid=(_N // _RT, _L // _BL),
        in_specs=[
            pl.BlockSpec(memory_space=pltpu.SMEM),
            pl.BlockSpec(memory_space=pltpu.SMEM),
            pl.BlockSpec((_RT, _BL), lambda i, j: (i, j)),
            pl.BlockSpec((_RT, _BL), lambda i, j: (i, j)),
        ],
        out_specs=[
            pl.BlockSpec((_RT, _BL, _K), lambda i, j: (i, j, 0)),
            pl.BlockSpec((_RT, _BL), lambda i, j: (i, j)),
        ],
        out_shape=[
            jax.ShapeDtypeStruct((_N, _L, _K), jnp.float32),
            jax.ShapeDtypeStruct((_N, _L), jnp.int32),
        ],
        compiler_params=pltpu.CompilerParams(
            dimension_semantics=("arbitrary", "arbitrary"),
        ),
    )(t.astype(jnp.int32), alpha_bars, x_0, m_i32)
    return c_t, x_t


# speedup vs baseline: 2.1960x; 2.1960x over previous
import jax
import jax.numpy as jnp
import numpy as np
from jax.experimental import pallas as pl
from jax.experimental.pallas import tpu as pltpu

_N, _L, _K = 128, 8192, 20
_RT = 8
_BL = 1024
_SG = _BL // 128
_TINY = np.float32(np.finfo(np.float32).tiny)
_L0 = np.float32(np.log(np.float64(np.float32(1e-8))))


def _threefry_bits(cnt):
    ks = (np.uint32(0), np.uint32(1), np.uint32(0x1BD11BDB))
    rot = (13, 15, 26, 6, 17, 29, 16, 24)

    x0 = jnp.zeros_like(cnt)
    x1 = cnt + ks[1]

    def rotl(v, d):
        return jax.lax.shift_left(v, np.uint32(d)) | jax.lax.shift_right_logical(
            v, np.uint32(32 - d))

    for i in range(5):
        rs = rot[:4] if i % 2 == 0 else rot[4:]
        for r in rs:
            x0 = x0 + x1
            x1 = rotl(x1, r)
            x1 = x0 ^ x1
        x0 = x0 + ks[(i + 1) % 3]
        x1 = x1 + ks[(i + 2) % 3] + np.uint32(i + 1)
    return x0 ^ x1


def _fused_kernel(t_ref, ab_ref, x0_ref, m_ref, c_ref, xt_ref):
    i0 = pl.program_id(0)
    i1 = pl.program_id(1)

    kio = jax.lax.broadcasted_iota(jnp.int32, (_K, _RT, 128), 0)
    rio = jax.lax.broadcasted_iota(jnp.int32, (1, _RT, 128), 1)
    cnt0 = (kio.astype(jnp.uint32)
            + jax.lax.broadcasted_iota(jnp.uint32, (_K, _RT, 128), 1)
            * np.uint32(_L * _K)
            + jax.lax.broadcasted_iota(jnp.uint32, (_K, _RT, 128), 2)
            * np.uint32(_K))

    a_v = jnp.zeros((1, _RT, 128), jnp.float32)
    q_v = jnp.zeros((1, _RT, 128), jnp.float32)
    for rr in range(_RT):
        n = i0 * _RT + rr
        ab = ab_ref[t_ref[n]]
        q = (1.0 - ab) / 20.0
        a = ab + q
        sel = rio == rr
        a_v = jnp.where(sel, a, a_v)
        q_v = jnp.where(sel, q, q_v)
    la_v = jnp.log(a_v + 1e-8)
    lq_v = jnp.log(q_v + 1e-8)

    base = jnp.uint32((i0 * _RT) * (_L * _K) + (i1 * _BL) * _K)

    for s in range(_SG):
        x0s = jnp.broadcast_to(x0_ref[:, s * 128:(s + 1) * 128][None],
                               (_K, _RT, 128))
        ms = jnp.broadcast_to(m_ref[:, s * 128:(s + 1) * 128][None],
                              (_K, _RT, 128)) != 0
        oh = x0s == kio
        c_like = jnp.where(ms, jnp.where(oh, a_v, q_v),
                           jnp.where(oh, 1.0, 0.0)).astype(jnp.float32)
        c_ref[:, :, s * 128:(s + 1) * 128] = c_like

        logits = jnp.where(ms, jnp.where(oh, la_v, lq_v),
                           jnp.where(oh, 0.0, _L0)).astype(jnp.float32)

        cnt = (base + np.uint32(s * 128 * _K)) + cnt0
        bits = _threefry_bits(cnt)
        fb = jax.lax.shift_right_logical(bits, np.uint32(9)) \
            | np.uint32(0x3F800000)
        f = jax.lax.bitcast_convert_type(fb, jnp.float32) - 1.0
        u = jnp.maximum(_TINY, f + _TINY)
        g = -jnp.log(-jnp.log(u))

        s_val = logits + g
        xt_ref[:, s * 128:(s + 1) * 128] = \
            jnp.argmax(s_val, axis=0).astype(jnp.int32)


@jax.jit
def kernel(x_0, mask_generate, t, alpha_bars):
    m_i32 = mask_generate.astype(jnp.int32)

    c_knl, x_t = pl.pallas_call(
        _fused_kernel,
        grid=(_N // _RT, _L // _BL),
        in_specs=[
            pl.BlockSpec(memory_space=pltpu.SMEM),
            pl.BlockSpec(memory_space=pltpu.SMEM),
            pl.BlockSpec((_RT, _BL), lambda i, j: (i, j)),
            pl.BlockSpec((_RT, _BL), lambda i, j: (i, j)),
        ],
        out_specs=[
            pl.BlockSpec((_K, _RT, _BL), lambda i, j: (0, i, j)),
            pl.BlockSpec((_RT, _BL), lambda i, j: (i, j)),
        ],
        out_shape=[
            jax.ShapeDtypeStruct((_K, _N, _L), jnp.float32),
            jax.ShapeDtypeStruct((_N, _L), jnp.int32),
        ],
        compiler_params=pltpu.CompilerParams(
            dimension_semantics=("arbitrary", "arbitrary"),
        ),
    )(t.astype(jnp.int32), alpha_bars, x_0, m_i32)
    return jnp.transpose(c_knl, (1, 2, 0)), x_t


# speedup vs baseline: 2.2806x; 1.0385x over previous
import jax
import jax.numpy as jnp
import numpy as np
from jax.experimental import pallas as pl
from jax.experimental.pallas import tpu as pltpu

_N, _L, _K = 128, 8192, 20
_RT = 8
_BL = 2048
_SG = _BL // 128
_TINY = np.float32(np.finfo(np.float32).tiny)
_L0 = np.float32(np.log(np.float64(np.float32(1e-8))))


def _threefry_bits(cnt):
    ks = (np.uint32(0), np.uint32(1), np.uint32(0x1BD11BDB))
    rot = (13, 15, 26, 6, 17, 29, 16, 24)

    x0 = jnp.zeros_like(cnt)
    x1 = cnt + ks[1]

    def rotl(v, d):
        return jax.lax.shift_left(v, np.uint32(d)) | jax.lax.shift_right_logical(
            v, np.uint32(32 - d))

    for i in range(5):
        rs = rot[:4] if i % 2 == 0 else rot[4:]
        for r in rs:
            x0 = x0 + x1
            x1 = rotl(x1, r)
            x1 = x0 ^ x1
        x0 = x0 + ks[(i + 1) % 3]
        x1 = x1 + ks[(i + 2) % 3] + np.uint32(i + 1)
    return x0 ^ x1


def _fused_kernel(t_ref, ab_ref, x0_ref, m_ref, c_ref, xt_ref):
    i0 = pl.program_id(0)
    i1 = pl.program_id(1)

    kio = jax.lax.broadcasted_iota(jnp.int32, (_K, _RT, 128), 0)
    rio = jax.lax.broadcasted_iota(jnp.int32, (1, _RT, 128), 1)
    cnt0 = (kio.astype(jnp.uint32)
            + jax.lax.broadcasted_iota(jnp.uint32, (_K, _RT, 128), 1)
            * np.uint32(_L * _K)
            + jax.lax.broadcasted_iota(jnp.uint32, (_K, _RT, 128), 2)
            * np.uint32(_K))

    a_v = jnp.zeros((1, _RT, 128), jnp.float32)
    q_v = jnp.zeros((1, _RT, 128), jnp.float32)
    for rr in range(_RT):
        n = i0 * _RT + rr
        ab = ab_ref[t_ref[n]]
        q = (1.0 - ab) / 20.0
        a = ab + q
        sel = rio == rr
        a_v = jnp.where(sel, a, a_v)
        q_v = jnp.where(sel, q, q_v)
    la_v = jnp.log(a_v + 1e-8)
    lq_v = jnp.log(q_v + 1e-8)

    base = jnp.uint32((i0 * _RT) * (_L * _K) + (i1 * _BL) * _K)

    for s in range(_SG):
        x0s = jnp.broadcast_to(x0_ref[:, s * 128:(s + 1) * 128][None],
                               (_K, _RT, 128))
        ms = m_ref[:, s * 128:(s + 1) * 128][None] != 0
        oh = x0s == kio
        hi = jnp.where(ms, a_v, 1.0)
        lo = jnp.where(ms, q_v, 0.0)
        c_like = jnp.where(oh, hi, lo).astype(jnp.float32)
        c_ref[:, :, s * 128:(s + 1) * 128] = c_like

        lhi = jnp.where(ms, la_v, 0.0)
        llo = jnp.where(ms, lq_v, _L0)
        logits = jnp.where(oh, lhi, llo).astype(jnp.float32)

        cnt = (base + np.uint32(s * 128 * _K)) + cnt0
        bits = _threefry_bits(cnt)
        fb = jax.lax.shift_right_logical(bits, np.uint32(9)) \
            | np.uint32(0x3F800000)
        f = jax.lax.bitcast_convert_type(fb, jnp.float32) - 1.0
        u = jnp.maximum(_TINY, f + _TINY)
        g = -jnp.log(-jnp.log(u))

        s_val = logits + g
        xt_ref[:, s * 128:(s + 1) * 128] = \
            jnp.argmax(s_val, axis=0).astype(jnp.int32)


@jax.jit
def kernel(x_0, mask_generate, t, alpha_bars):
    m_i32 = mask_generate.astype(jnp.int32)

    c_knl, x_t = pl.pallas_call(
        _fused_kernel,
        grid=(_N // _RT, _L // _BL),
        in_specs=[
            pl.BlockSpec(memory_space=pltpu.SMEM),
            pl.BlockSpec(memory_space=pltpu.SMEM),
            pl.BlockSpec((_RT, _BL), lambda i, j: (i, j)),
            pl.BlockSpec((_RT, _BL), lambda i, j: (i, j)),
        ],
        out_specs=[
            pl.BlockSpec((_K, _RT, _BL), lambda i, j: (0, i, j)),
            pl.BlockSpec((_RT, _BL), lambda i, j: (i, j)),
        ],
        out_shape=[
            jax.ShapeDtypeStruct((_K, _N, _L), jnp.float32),
            jax.ShapeDtypeStruct((_N, _L), jnp.int32),
        ],
        compiler_params=pltpu.CompilerParams(
            dimension_semantics=("arbitrary", "arbitrary"),
        ),
    )(t.astype(jnp.int32), alpha_bars, x_0, m_i32)
    return jnp.transpose(c_knl, (1, 2, 0)), x_t


# speedup vs baseline: 2.2950x; 1.0063x over previous
import jax
import jax.numpy as jnp
import numpy as np
from jax.experimental import pallas as pl
from jax.experimental.pallas import tpu as pltpu

_N, _L, _K = 128, 8192, 20
_RT = 8
_BL = 4096
_SG = _BL // 128
_TINY = np.float32(np.finfo(np.float32).tiny)
_L0 = np.float32(np.log(np.float64(np.float32(1e-8))))


def _threefry_bits(cnt):
    ks = (np.uint32(0), np.uint32(1), np.uint32(0x1BD11BDB))
    rot = (13, 15, 26, 6, 17, 29, 16, 24)

    x0 = jnp.zeros_like(cnt)
    x1 = cnt + ks[1]

    def rotl(v, d):
        return jax.lax.shift_left(v, np.uint32(d)) | jax.lax.shift_right_logical(
            v, np.uint32(32 - d))

    for i in range(5):
        rs = rot[:4] if i % 2 == 0 else rot[4:]
        for r in rs:
            x0 = x0 + x1
            x1 = rotl(x1, r)
            x1 = x0 ^ x1
        x0 = x0 + ks[(i + 1) % 3]
        x1 = x1 + ks[(i + 2) % 3] + np.uint32(i + 1)
    return x0 ^ x1


def _fused_kernel(t_ref, ab_ref, x0_ref, m_ref, c_ref, xt_ref):
    i0 = pl.program_id(0)
    i1 = pl.program_id(1)

    kio = jax.lax.broadcasted_iota(jnp.int32, (_K, _RT, 128), 0)
    rio = jax.lax.broadcasted_iota(jnp.int32, (1, _RT, 128), 1)
    cnt0 = (kio.astype(jnp.uint32)
            + jax.lax.broadcasted_iota(jnp.uint32, (_K, _RT, 128), 1)
            * np.uint32(_L * _K)
            + jax.lax.broadcasted_iota(jnp.uint32, (_K, _RT, 128), 2)
            * np.uint32(_K))

    a_v = jnp.zeros((1, _RT, 128), jnp.float32)
    q_v = jnp.zeros((1, _RT, 128), jnp.float32)
    for rr in range(_RT):
        n = i0 * _RT + rr
        ab = ab_ref[t_ref[n]]
        q = (1.0 - ab) / 20.0
        a = ab + q
        sel = rio == rr
        a_v = jnp.where(sel, a, a_v)
        q_v = jnp.where(sel, q, q_v)
    la_v = jnp.log(a_v + 1e-8)
    lq_v = jnp.log(q_v + 1e-8)

    base = jnp.uint32((i0 * _RT) * (_L * _K) + (i1 * _BL) * _K)

    for s in range(_SG):
        x0s = jnp.broadcast_to(x0_ref[:, s * 128:(s + 1) * 128][None],
                               (_K, _RT, 128))
        ms = m_ref[:, s * 128:(s + 1) * 128][None] != 0
        oh = x0s == kio
        hi = jnp.where(ms, a_v, 1.0)
        lo = jnp.where(ms, q_v, 0.0)
        c_like = jnp.where(oh, hi, lo).astype(jnp.float32)
        c_ref[:, :, s * 128:(s + 1) * 128] = c_like

        lhi = jnp.where(ms, la_v, 0.0)
        llo = jnp.where(ms, lq_v, _L0)
        logits = jnp.where(oh, lhi, llo).astype(jnp.float32)

        cnt = (base + np.uint32(s * 128 * _K)) + cnt0
        bits = _threefry_bits(cnt)
        fb = jax.lax.shift_right_logical(bits, np.uint32(9)) \
            | np.uint32(0x3F800000)
        f = jax.lax.bitcast_convert_type(fb, jnp.float32) - 1.0
        u = jnp.maximum(_TINY, f + _TINY)
        g = -jnp.log(-jnp.log(u))

        s_val = logits + g
        xt_ref[:, s * 128:(s + 1) * 128] = \
            jnp.argmax(s_val, axis=0).astype(jnp.int32)


@jax.jit
def kernel(x_0, mask_generate, t, alpha_bars):
    m_i32 = mask_generate.astype(jnp.int32)

    c_knl, x_t = pl.pallas_call(
        _fused_kernel,
        grid=(_N // _RT, _L // _BL),
        in_specs=[
            pl.BlockSpec(memory_space=pltpu.SMEM),
            pl.BlockSpec(memory_space=pltpu.SMEM),
            pl.BlockSpec((_RT, _BL), lambda i, j: (i, j)),
            pl.BlockSpec((_RT, _BL), lambda i, j: (i, j)),
        ],
        out_specs=[
            pl.BlockSpec((_K, _RT, _BL), lambda i, j: (0, i, j)),
            pl.BlockSpec((_RT, _BL), lambda i, j: (i, j)),
        ],
        out_shape=[
            jax.ShapeDtypeStruct((_K, _N, _L), jnp.float32),
            jax.ShapeDtypeStruct((_N, _L), jnp.int32),
        ],
        compiler_params=pltpu.CompilerParams(
            dimension_semantics=("arbitrary", "arbitrary"),
        ),
    )(t.astype(jnp.int32), alpha_bars, x_0, m_i32)
    return jnp.transpose(c_knl, (1, 2, 0)), x_t


# speedup vs baseline: 2.2964x; 1.0006x over previous
import jax
import jax.numpy as jnp
import numpy as np
from jax.experimental import pallas as pl
from jax.experimental.pallas import tpu as pltpu

_N, _L, _K = 128, 8192, 20
_RT = 8
_BL = 8192
_SG = _BL // 128
_TINY = np.float32(np.finfo(np.float32).tiny)
_L0 = np.float32(np.log(np.float64(np.float32(1e-8))))


def _threefry_bits(cnt):
    ks = (np.uint32(0), np.uint32(1), np.uint32(0x1BD11BDB))
    rot = (13, 15, 26, 6, 17, 29, 16, 24)

    x0 = jnp.zeros_like(cnt)
    x1 = cnt + ks[1]

    def rotl(v, d):
        return jax.lax.shift_left(v, np.uint32(d)) | jax.lax.shift_right_logical(
            v, np.uint32(32 - d))

    for i in range(5):
        rs = rot[:4] if i % 2 == 0 else rot[4:]
        for r in rs:
            x0 = x0 + x1
            x1 = rotl(x1, r)
            x1 = x0 ^ x1
        x0 = x0 + ks[(i + 1) % 3]
        x1 = x1 + ks[(i + 2) % 3] + np.uint32(i + 1)
    return x0 ^ x1


def _fused_kernel(t_ref, ab_ref, x0_ref, m_ref, c_ref, xt_ref):
    i0 = pl.program_id(0)
    i1 = pl.program_id(1)

    kio = jax.lax.broadcasted_iota(jnp.int32, (_K, _RT, 128), 0)
    rio = jax.lax.broadcasted_iota(jnp.int32, (1, _RT, 128), 1)
    cnt0 = (kio.astype(jnp.uint32)
            + jax.lax.broadcasted_iota(jnp.uint32, (_K, _RT, 128), 1)
            * np.uint32(_L * _K)
            + jax.lax.broadcasted_iota(jnp.uint32, (_K, _RT, 128), 2)
            * np.uint32(_K))

    a_v = jnp.zeros((1, _RT, 128), jnp.float32)
    q_v = jnp.zeros((1, _RT, 128), jnp.float32)
    for rr in range(_RT):
        n = i0 * _RT + rr
        ab = ab_ref[t_ref[n]]
        q = (1.0 - ab) / 20.0
        a = ab + q
        sel = rio == rr
        a_v = jnp.where(sel, a, a_v)
        q_v = jnp.where(sel, q, q_v)
    la_v = jnp.log(a_v + 1e-8)
    lq_v = jnp.log(q_v + 1e-8)

    base = jnp.uint32((i0 * _RT) * (_L * _K) + (i1 * _BL) * _K)

    for s in range(_SG):
        x0s = jnp.broadcast_to(x0_ref[:, s * 128:(s + 1) * 128][None],
                               (_K, _RT, 128))
        ms = m_ref[:, s * 128:(s + 1) * 128][None] != 0
        oh = x0s == kio
        hi = jnp.where(ms, a_v, 1.0)
        lo = jnp.where(ms, q_v, 0.0)
        c_like = jnp.where(oh, hi, lo).astype(jnp.float32)
        c_ref[:, :, s * 128:(s + 1) * 128] = c_like

        lhi = jnp.where(ms, la_v, 0.0)
        llo = jnp.where(ms, lq_v, _L0)
        logits = jnp.where(oh, lhi, llo).astype(jnp.float32)

        cnt = (base + np.uint32(s * 128 * _K)) + cnt0
        bits = _threefry_bits(cnt)
        fb = jax.lax.shift_right_logical(bits, np.uint32(9)) \
            | np.uint32(0x3F800000)
        f = jax.lax.bitcast_convert_type(fb, jnp.float32) - 1.0
        u = jnp.maximum(_TINY, f + _TINY)
        g = -jnp.log(-jnp.log(u))

        s_val = logits + g
        xt_ref[:, s * 128:(s + 1) * 128] = \
            jnp.argmax(s_val, axis=0).astype(jnp.int32)


@jax.jit
def kernel(x_0, mask_generate, t, alpha_bars):
    m_i32 = mask_generate.astype(jnp.int32)

    c_knl, x_t = pl.pallas_call(
        _fused_kernel,
        grid=(_N // _RT, _L // _BL),
        in_specs=[
            pl.BlockSpec(memory_space=pltpu.SMEM),
            pl.BlockSpec(memory_space=pltpu.SMEM),
            pl.BlockSpec((_RT, _BL), lambda i, j: (i, j)),
            pl.BlockSpec((_RT, _BL), lambda i, j: (i, j)),
        ],
        out_specs=[
            pl.BlockSpec((_K, _RT, _BL), lambda i, j: (0, i, j)),
            pl.BlockSpec((_RT, _BL), lambda i, j: (i, j)),
        ],
        out_shape=[
            jax.ShapeDtypeStruct((_K, _N, _L), jnp.float32),
            jax.ShapeDtypeStruct((_N, _L), jnp.int32),
        ],
        compiler_params=pltpu.CompilerParams(
            dimension_semantics=("arbitrary", "arbitrary"),
        ),
    )(t.astype(jnp.int32), alpha_bars, x_0, m_i32)
    return jnp.transpose(c_knl, (1, 2, 0)), x_t
